# baseline (speedup 1.0000x reference)
import functools

import jax
import jax.numpy as jnp
from jax import lax
from jax.experimental import pallas as pl
from jax.experimental.pallas import tpu as pltpu
from jax.experimental.pallas import tpu_sc as plsc

BN_EPS = 1e-5
EDGE_CHUNK = 96
NBUF = 4
TRASH_ROWS = 8


def _sc_counts():
    try:
        info = plsc.get_sparse_core_info()
        return info.num_cores, info.num_subcores
    except Exception:
        return 2, 16


def _pad_edges(src, dst, n_nodes, nw, chunk):
    n_edges = src.shape[0]
    ept = n_edges // nw
    ept_pad = -(-ept // chunk) * chunk
    npad = ept_pad - ept
    if npad == 0:
        return src, dst, ept
    tile_i = jax.lax.broadcasted_iota(jnp.int32, (nw, npad), 0)
    pad_i = jax.lax.broadcasted_iota(jnp.int32, (nw, npad), 1)
    pad_src = (tile_i * 613 + pad_i * 97) % n_nodes
    pad_dst = n_nodes + (tile_i + pad_i) % TRASH_ROWS
    src2 = jnp.concatenate([src.reshape(nw, ept), pad_src], axis=1).reshape(-1)
    dst2 = jnp.concatenate([dst.reshape(nw, ept), pad_dst], axis=1).reshape(-1)
    return src2, dst2, ept_pad


def _make_agg(n_nodes, d, e_per_tile):
    NC, NS = _sc_counts()
    chunk = EDGE_CHUNK
    assert e_per_tile % chunk == 0
    n_steps = e_per_tile // chunk
    acc_rows = n_nodes + TRASH_ROWS
    rows_per_tile = (n_nodes // NS) // 8 * 8
    rem_rows = n_nodes - rows_per_tile * NS

    mesh = plsc.VectorSubcoreMesh(core_axis_name="c", subcore_axis_name="s")

    rows_t = [pltpu.VMEM((chunk, d), jnp.float32) for _ in range(NBUF)]
    srcb_t = [pltpu.VMEM((chunk,), jnp.int32) for _ in range(NBUF)]
    dstb_t = [pltpu.VMEM((chunk,), jnp.int32) for _ in range(NBUF)]
    sems = [pltpu.SemaphoreType.DMA for _ in range(4 * NBUF)]

    @functools.partial(
        pl.kernel,
        out_type=jax.ShapeDtypeStruct((NC * n_nodes, d), jnp.float32),
        mesh=mesh,
        scratch_types=[pltpu.VMEM_SHARED((acc_rows, d), jnp.float32)]
        + rows_t + srcb_t + dstb_t + sems,
    )
    def agg(x_hbm, src_hbm, dst_hbm, out_hbm, acc_ref, *bufs):
        rows = bufs[0:NBUF]
        srcb = bufs[NBUF:2 * NBUF]
        dstb = bufs[2 * NBUF:3 * NBUF]
        gsem = bufs[3 * NBUF:4 * NBUF]
        psem = bufs[4 * NBUF:5 * NBUF]
        dsem = bufs[5 * NBUF:6 * NBUF]
        ssem = bufs[6 * NBUF:7 * NBUF]
        cid = lax.axis_index("c")
        sid = lax.axis_index("s")
        wid = cid * NS + sid
        ebase = wid * e_per_tile

        r0 = sid * rows_per_tile
        pltpu.sync_copy(
            x_hbm.at[pl.ds(r0, rows_per_tile)],
            acc_ref.at[pl.ds(r0, rows_per_tile)],
        )
        if rem_rows:
            @pl.when(sid == NS - 1)
            def _():
                pltpu.sync_copy(
                    x_hbm.at[pl.ds(NS * rows_per_tile, rem_rows)],
                    acc_ref.at[pl.ds(NS * rows_per_tile, rem_rows)],
                )
        plsc.subcore_barrier()

        def echunk(hbm, k):
            return hbm.at[pl.ds(ebase + pl.multiple_of(k * chunk, 8), chunk)]

        def idx_start(k, j):
            pltpu.async_copy(echunk(src_hbm, k), srcb[j], psem[j])
            pltpu.async_copy(echunk(dst_hbm, k), dstb[j], dsem[j])

        def idx_wait(j):
            pltpu.make_async_copy(src_hbm.at[pl.ds(0, chunk)], srcb[j],
                                  psem[j]).wait()
            pltpu.make_async_copy(dst_hbm.at[pl.ds(0, chunk)], dstb[j],
                                  dsem[j]).wait()

        def gather_start(j):
            pltpu.async_copy(x_hbm.at[srcb[j]], rows[j], gsem[j])

        def gather_wait(j):
            pltpu.make_async_copy(x_hbm.at[srcb[j]], rows[j], gsem[j]).wait()

        def scatter_start(j):
            pltpu.async_copy(rows[j], acc_ref.at[dstb[j]], ssem[j], add=True)

        def scatter_wait(j):
            pltpu.make_async_copy(rows[j], acc_ref.at[dstb[j]], ssem[j]).wait()

        for k in range(min(NBUF, n_steps)):
            idx_start(k, k)
        for k in range(min(NBUF, n_steps)):
            idx_wait(k)
            gather_start(k)

        def body(m, carry):
            for j in range(NBUF):
                k = NBUF * m + j
                gather_wait(j)
                scatter_start(j)

                @pl.when(k + NBUF < n_steps)
                def _():
                    idx_start(k + NBUF, j)
                    scatter_wait(j)
                    idx_wait(j)
                    gather_start(j)
            return carry

        lax.fori_loop(0, n_steps // NBUF, body, 0)
        for k in range((n_steps // NBUF) * NBUF, n_steps):
            j = k % NBUF
            gather_wait(j)
            scatter_start(j)
        for j in range(min(NBUF, n_steps)):
            scatter_wait(j)
        plsc.subcore_barrier()

        o0 = cid * n_nodes + r0
        pltpu.sync_copy(
            acc_ref.at[pl.ds(r0, rows_per_tile)],
            out_hbm.at[pl.ds(o0, rows_per_tile)],
        )
        if rem_rows:
            @pl.when(sid == NS - 1)
            def _():
                pltpu.sync_copy(
                    acc_ref.at[pl.ds(NS * rows_per_tile, rem_rows)],
                    out_hbm.at[pl.ds(cid * n_nodes + NS * rows_per_tile, rem_rows)],
                )

    return agg, NC


def _make_dense(n_nodes, d_in, d_out, nc):
    def body(p_ref, x_ref, w_ref, b_ref, g_ref, be_ref, o_ref):
        agg = p_ref[0:n_nodes, :]
        for c in range(1, nc):
            agg = agg + p_ref[c * n_nodes:(c + 1) * n_nodes, :]
        agg = agg - (nc - 1) * x_ref[...]
        h = jnp.dot(agg, w_ref[...], preferred_element_type=jnp.float32)
        h = h + b_ref[...]
        mu = jnp.mean(h, axis=0, keepdims=True)
        var = jnp.mean((h - mu) ** 2, axis=0, keepdims=True)
        h = (h - mu) * lax.rsqrt(var + BN_EPS)
        h = h * g_ref[...] + be_ref[...]
        o_ref[...] = jnp.maximum(h, 0.0)

    return pl.pallas_call(
        body,
        out_shape=jax.ShapeDtypeStruct((n_nodes, d_out), jnp.float32),
    )


def kernel(node_feat, edge_index, W0, b0, gamma0, beta0, W1, b1, gamma1, beta1):
    n, d_in = node_feat.shape
    n_edges = edge_index.shape[1]
    nc, ns = _sc_counts()
    nw = nc * ns
    src = edge_index[0].astype(jnp.int32)
    dst = edge_index[1].astype(jnp.int32)
    src, dst, ept = _pad_edges(src, dst, n, nw, EDGE_CHUNK)

    agg0, _ = _make_agg(n, d_in, ept)
    dense0 = _make_dense(n, d_in, W0.shape[1], nc)
    p = agg0(node_feat, src, dst)
    h0 = dense0(p, node_feat, W0, b0.reshape(1, -1), gamma0.reshape(1, -1),
                beta0.reshape(1, -1))

    agg1 = _make_agg(n, W0.shape[1], ept)[0]
    dense1 = _make_dense(n, W0.shape[1], W1.shape[1], nc)
    q = agg1(h0, src, dst)
    h1 = dense1(q, h0, W1, b1.reshape(1, -1), gamma1.reshape(1, -1),
                beta1.reshape(1, -1))
    return h1


# speedup vs baseline: 1.0045x; 1.0045x over previous
import functools

import jax
import jax.numpy as jnp
from jax import lax
from jax.experimental import pallas as pl
from jax.experimental.pallas import tpu as pltpu
from jax.experimental.pallas import tpu_sc as plsc

BN_EPS = 1e-5
EDGE_CHUNK = 128
NBUF = 3
TRASH_ROWS = 8


def _sc_counts():
    try:
        info = plsc.get_sparse_core_info()
        return info.num_cores, info.num_subcores
    except Exception:
        return 2, 16


def _pad_edges(src, dst, n_nodes, nw, chunk):
    n_edges = src.shape[0]
    ept = n_edges // nw
    ept_pad = -(-ept // chunk) * chunk
    npad = ept_pad - ept
    if npad == 0:
        return src, dst, ept
    tile_i = jax.lax.broadcasted_iota(jnp.int32, (nw, npad), 0)
    pad_i = jax.lax.broadcasted_iota(jnp.int32, (nw, npad), 1)
    pad_src = (tile_i * 613 + pad_i * 97) % n_nodes
    pad_dst = n_nodes + (tile_i + pad_i) % TRASH_ROWS
    src2 = jnp.concatenate([src.reshape(nw, ept), pad_src], axis=1).reshape(-1)
    dst2 = jnp.concatenate([dst.reshape(nw, ept), pad_dst], axis=1).reshape(-1)
    return src2, dst2, ept_pad


def _make_agg(n_nodes, d, e_per_tile):
    NC, NS = _sc_counts()
    chunk = EDGE_CHUNK
    assert e_per_tile % chunk == 0
    n_steps = e_per_tile // chunk
    acc_rows = n_nodes + TRASH_ROWS
    rows_per_tile = (n_nodes // NS) // 8 * 8
    rem_rows = n_nodes - rows_per_tile * NS

    mesh = plsc.VectorSubcoreMesh(core_axis_name="c", subcore_axis_name="s")

    rows_t = [pltpu.VMEM((chunk, d), jnp.float32) for _ in range(NBUF)]
    srcb_t = [pltpu.VMEM((chunk,), jnp.int32) for _ in range(NBUF)]
    dstb_t = [pltpu.VMEM((chunk,), jnp.int32) for _ in range(NBUF)]
    sems = [pltpu.SemaphoreType.DMA for _ in range(4 * NBUF)]

    @functools.partial(
        pl.kernel,
        out_type=jax.ShapeDtypeStruct((NC * n_nodes, d), jnp.float32),
        mesh=mesh,
        scratch_types=[pltpu.VMEM_SHARED((acc_rows, d), jnp.float32)]
        + rows_t + srcb_t + dstb_t + sems,
    )
    def agg(x_hbm, src_hbm, dst_hbm, out_hbm, acc_ref, *bufs):
        rows = bufs[0:NBUF]
        srcb = bufs[NBUF:2 * NBUF]
        dstb = bufs[2 * NBUF:3 * NBUF]
        gsem = bufs[3 * NBUF:4 * NBUF]
        psem = bufs[4 * NBUF:5 * NBUF]
        dsem = bufs[5 * NBUF:6 * NBUF]
        ssem = bufs[6 * NBUF:7 * NBUF]
        cid = lax.axis_index("c")
        sid = lax.axis_index("s")
        wid = cid * NS + sid
        ebase = wid * e_per_tile

        r0 = sid * rows_per_tile
        pltpu.sync_copy(
            x_hbm.at[pl.ds(r0, rows_per_tile)],
            acc_ref.at[pl.ds(r0, rows_per_tile)],
        )
        if rem_rows:
            @pl.when(sid == NS - 1)
            def _():
                pltpu.sync_copy(
                    x_hbm.at[pl.ds(NS * rows_per_tile, rem_rows)],
                    acc_ref.at[pl.ds(NS * rows_per_tile, rem_rows)],
                )
        plsc.subcore_barrier()

        def echunk(hbm, k):
            return hbm.at[pl.ds(ebase + pl.multiple_of(k * chunk, 8), chunk)]

        def idx_start(k, j):
            pltpu.async_copy(echunk(src_hbm, k), srcb[j], psem[j])
            pltpu.async_copy(echunk(dst_hbm, k), dstb[j], dsem[j])

        def idx_wait(j):
            pltpu.make_async_copy(src_hbm.at[pl.ds(0, chunk)], srcb[j],
                                  psem[j]).wait()
            pltpu.make_async_copy(dst_hbm.at[pl.ds(0, chunk)], dstb[j],
                                  dsem[j]).wait()

        def gather_start(j):
            pltpu.async_copy(x_hbm.at[srcb[j]], rows[j], gsem[j])

        def gather_wait(j):
            pltpu.make_async_copy(x_hbm.at[srcb[j]], rows[j], gsem[j]).wait()

        def scatter_start(j):
            pltpu.async_copy(rows[j], acc_ref.at[dstb[j]], ssem[j], add=True)

        def scatter_wait(j):
            pltpu.make_async_copy(rows[j], acc_ref.at[dstb[j]], ssem[j]).wait()

        for k in range(min(NBUF, n_steps)):
            idx_start(k, k)
        for k in range(min(NBUF, n_steps)):
            idx_wait(k)
            gather_start(k)

        def body(m, carry):
            for j in range(NBUF):
                k = NBUF * m + j
                gather_wait(j)
                scatter_start(j)

                @pl.when(k + NBUF < n_steps)
                def _():
                    idx_start(k + NBUF, j)
                    scatter_wait(j)
                    idx_wait(j)
                    gather_start(j)
            return carry

        lax.fori_loop(0, n_steps // NBUF, body, 0)
        for k in range((n_steps // NBUF) * NBUF, n_steps):
            j = k % NBUF
            gather_wait(j)
            scatter_start(j)
        for j in range(min(NBUF, n_steps)):
            scatter_wait(j)
        plsc.subcore_barrier()

        o0 = cid * n_nodes + r0
        pltpu.sync_copy(
            acc_ref.at[pl.ds(r0, rows_per_tile)],
            out_hbm.at[pl.ds(o0, rows_per_tile)],
        )
        if rem_rows:
            @pl.when(sid == NS - 1)
            def _():
                pltpu.sync_copy(
                    acc_ref.at[pl.ds(NS * rows_per_tile, rem_rows)],
                    out_hbm.at[pl.ds(cid * n_nodes + NS * rows_per_tile, rem_rows)],
                )

    return agg, NC


def _make_dense(n_nodes, d_in, d_out, nc):
    def body(p_ref, x_ref, w_ref, b_ref, g_ref, be_ref, o_ref):
        agg = p_ref[0:n_nodes, :]
        for c in range(1, nc):
            agg = agg + p_ref[c * n_nodes:(c + 1) * n_nodes, :]
        agg = agg - (nc - 1) * x_ref[...]
        h = jnp.dot(agg, w_ref[...], preferred_element_type=jnp.float32)
        h = h + b_ref[...]
        mu = jnp.mean(h, axis=0, keepdims=True)
        var = jnp.mean((h - mu) ** 2, axis=0, keepdims=True)
        h = (h - mu) * lax.rsqrt(var + BN_EPS)
        h = h * g_ref[...] + be_ref[...]
        o_ref[...] = jnp.maximum(h, 0.0)

    return pl.pallas_call(
        body,
        out_shape=jax.ShapeDtypeStruct((n_nodes, d_out), jnp.float32),
    )


def kernel(node_feat, edge_index, W0, b0, gamma0, beta0, W1, b1, gamma1, beta1):
    n, d_in = node_feat.shape
    n_edges = edge_index.shape[1]
    nc, ns = _sc_counts()
    nw = nc * ns
    src = edge_index[0].astype(jnp.int32)
    dst = edge_index[1].astype(jnp.int32)
    src, dst, ept = _pad_edges(src, dst, n, nw, EDGE_CHUNK)

    agg0, _ = _make_agg(n, d_in, ept)
    dense0 = _make_dense(n, d_in, W0.shape[1], nc)
    p = agg0(node_feat, src, dst)
    h0 = dense0(p, node_feat, W0, b0.reshape(1, -1), gamma0.reshape(1, -1),
                beta0.reshape(1, -1))

    agg1 = _make_agg(n, W0.shape[1], ept)[0]
    dense1 = _make_dense(n, W0.shape[1], W1.shape[1], nc)
    q = agg1(h0, src, dst)
    h1 = dense1(q, h0, W1, b1.reshape(1, -1), gamma1.reshape(1, -1),
                beta1.reshape(1, -1))
    return h1
